# baseline (speedup 1.0000x reference)
import jax
import jax.numpy as jnp
from jax.experimental import pallas as pl
from jax.experimental.pallas import tpu as pltpu

_BS, _N, _D, _K = 8, 4096, 64, 512
_MAX_ITER = 8
_SEED = 123
_NCHUNK = 1024


def _init_centers_like_ref(x):
    key = jax.random.key(_SEED)
    keys = jax.random.split(key, _BS)

    def pick(xi, k):
        idx = jax.random.choice(k, _N, shape=(_K,), replace=False)
        return jnp.take(xi, idx, axis=0)

    return jax.vmap(pick)(x, keys)


def _kmeans_body(x_ref, c0_ref, labels_ref, centers_ref, inertia_ref):
    x = x_ref[0]
    x2 = jnp.sum(x * x, axis=1, keepdims=True)
    iota_k = jax.lax.broadcasted_iota(jnp.int32, (_NCHUNK, _K), 1)

    def dists_chunk(s, centers2, c2):
        xw = x[s * _NCHUNK:(s + 1) * _NCHUNK]
        xc2 = jnp.dot(xw, centers2.T, preferred_element_type=jnp.float32)
        d2 = jnp.maximum(
            (x2[s * _NCHUNK:(s + 1) * _NCHUNK] + c2) - xc2, 0.0)
        return xw, d2

    def body(_, centers):
        c2 = jnp.sum(centers * centers, axis=1)[None, :]
        centers2 = centers + centers
        sums = jnp.zeros((_K, _D), jnp.float32)
        counts = jnp.zeros((1, _K), jnp.float32)
        for s in range(_N // _NCHUNK):
            xw, d2 = dists_chunk(s, centers2, c2)
            labels = jnp.argmin(d2, axis=1)
            onehot = (labels[:, None] == iota_k).astype(jnp.float32)
            sums = sums + jax.lax.dot_general(
                onehot, xw, (((0,), (0,)), ((), ())),
                preferred_element_type=jnp.float32)
            counts = counts + jnp.sum(onehot, axis=0, keepdims=True)
        counts_t = counts.T
        new_centers = sums / jnp.maximum(counts_t, 1.0)
        return jnp.where(counts_t > 0, new_centers, centers)

    centers = jax.lax.fori_loop(0, _MAX_ITER, body, c0_ref[0])

    c2 = jnp.sum(centers * centers, axis=1)[None, :]
    centers2 = centers + centers
    acc = jnp.zeros((), jnp.float32)
    for s in range(_N // _NCHUNK):
        _, d2 = dists_chunk(s, centers2, c2)
        labels = jnp.argmin(d2, axis=1)
        labels_ref[0, 0, pl.ds(s * _NCHUNK, _NCHUNK)] = labels
        acc = acc + jnp.sum(jnp.min(d2, axis=1))
    centers_ref[0] = centers
    inertia_ref[...] = acc.reshape(1, 1, 1)


def kernel(x):
    c0 = _init_centers_like_ref(x)
    labels3, centers, inertia2 = pl.pallas_call(
        _kmeans_body,
        grid=(_BS,),
        in_specs=[
            pl.BlockSpec((1, _N, _D), lambda i: (i, 0, 0)),
            pl.BlockSpec((1, _K, _D), lambda i: (i, 0, 0)),
        ],
        out_specs=[
            pl.BlockSpec((1, 1, _N), lambda i: (i, 0, 0)),
            pl.BlockSpec((1, _K, _D), lambda i: (i, 0, 0)),
            pl.BlockSpec((1, 1, 1), lambda i: (i, 0, 0)),
        ],
        out_shape=[
            jax.ShapeDtypeStruct((_BS, 1, _N), jnp.int32),
            jax.ShapeDtypeStruct((_BS, _K, _D), jnp.float32),
            jax.ShapeDtypeStruct((_BS, 1, 1), jnp.float32),
        ],
        compiler_params=pltpu.CompilerParams(
            dimension_semantics=("parallel",)),
    )(x, c0)
    return labels3.reshape(_BS, _N), centers, inertia2.reshape(_BS)


# speedup vs baseline: 1.1010x; 1.1010x over previous
import jax
import jax.numpy as jnp
from jax.experimental import pallas as pl
from jax.experimental.pallas import tpu as pltpu

_BS, _N, _D, _K = 8, 4096, 64, 512
_MAX_ITER = 8
_SEED = 123
_NCHUNK = 1024


def _init_centers_like_ref(x):
    key = jax.random.key(_SEED)
    keys = jax.random.split(key, _BS)

    def pick(xi, k):
        idx = jax.random.choice(k, _N, shape=(_K,), replace=False)
        return jnp.take(xi, idx, axis=0)

    return jax.vmap(pick)(x, keys)


def _kmeans_body(x_ref, c0_ref, labels_ref, centers_ref, inertia_ref):
    x = x_ref[0]
    x2 = jnp.sum(x * x, axis=1, keepdims=True)
    iota_k = jax.lax.broadcasted_iota(jnp.int32, (_NCHUNK, _K), 1)

    def dists_chunk(s, centers2, c2):
        xw = x[s * _NCHUNK:(s + 1) * _NCHUNK]
        xc2 = jnp.dot(xw, centers2.T, preferred_element_type=jnp.float32)
        d2 = jnp.maximum(
            (x2[s * _NCHUNK:(s + 1) * _NCHUNK] + c2) - xc2, 0.0)
        return xw, d2

    ones_col = jnp.ones((_NCHUNK, 1), jnp.float32)

    def body(_, centers):
        c2 = jnp.sum(centers * centers, axis=1)[None, :]
        centers2 = centers + centers
        sums_aug = jnp.zeros((_K, _D + 1), jnp.float32)
        for s in range(_N // _NCHUNK):
            xw, d2 = dists_chunk(s, centers2, c2)
            d2min = jnp.min(d2, axis=1, keepdims=True)
            wi = jnp.where(d2 == d2min, iota_k, _K)
            labels = jnp.min(wi, axis=1, keepdims=True)
            onehot = (wi == labels).astype(jnp.float32)
            xw_aug = jnp.concatenate([xw, ones_col], axis=1)
            sums_aug = sums_aug + jax.lax.dot_general(
                onehot, xw_aug, (((0,), (0,)), ((), ())),
                preferred_element_type=jnp.float32)
        sums = sums_aug[:, :_D]
        counts = sums_aug[:, _D:]
        new_centers = sums / jnp.maximum(counts, 1.0)
        return jnp.where(counts > 0, new_centers, centers)

    centers = jax.lax.fori_loop(0, _MAX_ITER, body, c0_ref[0])

    c2 = jnp.sum(centers * centers, axis=1)[None, :]
    centers2 = centers + centers
    acc = jnp.zeros((), jnp.float32)
    for s in range(_N // _NCHUNK):
        _, d2 = dists_chunk(s, centers2, c2)
        d2min = jnp.min(d2, axis=1, keepdims=True)
        labels = jnp.min(jnp.where(d2 == d2min, iota_k, _K), axis=1)
        labels_ref[0, 0, pl.ds(s * _NCHUNK, _NCHUNK)] = labels
        acc = acc + jnp.sum(d2min)
    centers_ref[0] = centers
    inertia_ref[...] = acc.reshape(1, 1, 1)


def kernel(x):
    c0 = _init_centers_like_ref(x)
    labels3, centers, inertia2 = pl.pallas_call(
        _kmeans_body,
        grid=(_BS,),
        in_specs=[
            pl.BlockSpec((1, _N, _D), lambda i: (i, 0, 0)),
            pl.BlockSpec((1, _K, _D), lambda i: (i, 0, 0)),
        ],
        out_specs=[
            pl.BlockSpec((1, 1, _N), lambda i: (i, 0, 0)),
            pl.BlockSpec((1, _K, _D), lambda i: (i, 0, 0)),
            pl.BlockSpec((1, 1, 1), lambda i: (i, 0, 0)),
        ],
        out_shape=[
            jax.ShapeDtypeStruct((_BS, 1, _N), jnp.int32),
            jax.ShapeDtypeStruct((_BS, _K, _D), jnp.float32),
            jax.ShapeDtypeStruct((_BS, 1, 1), jnp.float32),
        ],
        compiler_params=pltpu.CompilerParams(
            dimension_semantics=("parallel",)),
    )(x, c0)
    return labels3.reshape(_BS, _N), centers, inertia2.reshape(_BS)
